# Initial kernel scaffold; baseline (speedup 1.0000x reference)
import jax
import jax.numpy as jnp
from jax.experimental import pallas as pl


def kernel(x, edge_index, edge_attr, batch, params):
    raise NotImplementedError("write your pallas kernel here")


# speedup vs baseline: 18.6703x; 18.6703x over previous
import functools

import jax
import jax.numpy as jnp
from jax import lax
from jax.experimental import pallas as pl
from jax.experimental.pallas import tpu as pltpu
from jax.experimental.pallas import tpu_sc as plsc

N = 10000; E = 320000; D = 128; DE = 16; H = 8; C = 8; HC = H * C; B = 64
NC_SC = 2
NS_SC = 16
NTILES = NC_SC * NS_SC
EPT = E // NTILES
CH = 80
NCHUNK = EPT // CH
NPT = 640
NP_PAD = NPT * NS_SC

_SC_MESH = plsc.VectorSubcoreMesh(core_axis_name="c", subcore_axis_name="s")
_SC_PARAMS = pltpu.CompilerParams(needs_layout_passes=False,
                                  use_tc_tiling_on_sc=False)


def _stats_body(x_ref, o_ref):
    x = x_ref[...]
    @pl.when(pl.program_id(0) == 0)
    def _():
        o_ref[...] = jnp.zeros_like(o_ref)
    o_ref[0:1, :] += jnp.sum(x, 0, keepdims=True)
    o_ref[1:2, :] += jnp.sum(x * x, 0, keepdims=True)


def _col_stats(x, blk):
    n, d = x.shape
    return pl.pallas_call(
        _stats_body,
        grid=(n // blk,),
        in_specs=[pl.BlockSpec((blk, d), lambda i: (i, 0))],
        out_specs=pl.BlockSpec((2, d), lambda i: (0, 0)),
        out_shape=jax.ShapeDtypeStruct((2, d), jnp.float32),
    )(x)


def _bn_from_stats(x, st, g, b, n):
    mean = st[0:1, :] / n
    var = st[1:2, :] / n - mean * mean
    return (x - mean) * jax.lax.rsqrt(var + 1e-5) * g[None, :] + b[None, :]


def _bn_mm_body(x_ref, st_ref, g_ref, b_ref, wl_ref, wr_ref, xl_ref, xr_ref):
    h = _bn_from_stats(x_ref[...], st_ref[...], g_ref[0], b_ref[0], float(N))
    xl_ref[...] = jnp.dot(h, wl_ref[...], preferred_element_type=jnp.float32)
    xr_ref[...] = jnp.dot(h, wr_ref[...], preferred_element_type=jnp.float32)


def _bn_mm1(x, st, g, b, wl, wr):
    blk = 2000
    return pl.pallas_call(
        _bn_mm_body,
        grid=(N // blk,),
        in_specs=[
            pl.BlockSpec((blk, D), lambda i: (i, 0)),
            pl.BlockSpec((2, D), lambda i: (0, 0)),
            pl.BlockSpec((1, D), lambda i: (0, 0)),
            pl.BlockSpec((1, D), lambda i: (0, 0)),
            pl.BlockSpec((D, HC), lambda i: (0, 0)),
            pl.BlockSpec((D, HC), lambda i: (0, 0)),
        ],
        out_specs=[
            pl.BlockSpec((blk, HC), lambda i: (i, 0)),
            pl.BlockSpec((blk, HC), lambda i: (i, 0)),
        ],
        out_shape=[
            jax.ShapeDtypeStruct((N, HC), jnp.float32),
            jax.ShapeDtypeStruct((N, HC), jnp.float32),
        ],
    )(x, st, g[None, :], b[None, :], wl, wr)


def _bn_ea_body(ea_ref, st_ref, g_ref, b_ref, o_ref):
    o_ref[...] = _bn_from_stats(ea_ref[...], st_ref[...], g_ref[0], b_ref[0],
                                float(E))


def _bn_ea(ea, st, g, b):
    blk = 8000
    return pl.pallas_call(
        _bn_ea_body,
        grid=(E // blk,),
        in_specs=[
            pl.BlockSpec((blk, DE), lambda i: (i, 0)),
            pl.BlockSpec((2, DE), lambda i: (0, 0)),
            pl.BlockSpec((1, DE), lambda i: (0, 0)),
            pl.BlockSpec((1, DE), lambda i: (0, 0)),
        ],
        out_specs=pl.BlockSpec((blk, DE), lambda i: (i, 0)),
        out_shape=jax.ShapeDtypeStruct((E, DE), jnp.float32),
    )(ea, st, g[None, :], b[None, :])


def _ee_mm_body(ea_ref, we_ref, o_ref):
    o_ref[...] = jnp.dot(ea_ref[...], we_ref[...],
                         preferred_element_type=jnp.float32)


def _ee_mm(ean, we):
    blk = 8000
    return pl.pallas_call(
        _ee_mm_body,
        grid=(E // blk,),
        in_specs=[
            pl.BlockSpec((blk, DE), lambda i: (i, 0)),
            pl.BlockSpec((DE, HC), lambda i: (0, 0)),
        ],
        out_specs=pl.BlockSpec((blk, HC), lambda i: (i, 0)),
        out_shape=jax.ShapeDtypeStruct((E, HC), jnp.float32),
    )(ean, we)


def _merge_mm_body(o0_ref, o1_ref, bias_ref, wl_ref, wr_ref,
                   xl_ref, xr_ref):
    h = o0_ref[...] + o1_ref[...] + bias_ref[...]
    h = jnp.maximum(h, 0.2 * h)
    xl_ref[...] = jnp.dot(h, wl_ref[...], preferred_element_type=jnp.float32)
    xr_ref[...] = jnp.dot(h, wr_ref[...], preferred_element_type=jnp.float32)


def _merge_mm(o0, o1, bias, wl, wr):
    blk = 2000
    return pl.pallas_call(
        _merge_mm_body,
        grid=(N // blk,),
        in_specs=[
            pl.BlockSpec((blk, HC), lambda i: (i, 0)),
            pl.BlockSpec((blk, HC), lambda i: (i, 0)),
            pl.BlockSpec((1, HC), lambda i: (0, 0)),
            pl.BlockSpec((HC, HC), lambda i: (0, 0)),
            pl.BlockSpec((HC, HC), lambda i: (0, 0)),
        ],
        out_specs=[
            pl.BlockSpec((blk, HC), lambda i: (i, 0)),
            pl.BlockSpec((blk, HC), lambda i: (i, 0)),
        ],
        out_shape=[
            jax.ShapeDtypeStruct((N, HC), jnp.float32),
            jax.ShapeDtypeStruct((N, HC), jnp.float32),
        ],
    )(o0, o1, bias[None, :], wl, wr)


def _merge_h_body(o0_ref, o1_ref, bias_ref, h_ref):
    h = o0_ref[...] + o1_ref[...] + bias_ref[...]
    h_ref[...] = jnp.maximum(h, 0.2 * h)


def _merge_h(o0, o1, bias):
    blk = 2048
    return pl.pallas_call(
        _merge_h_body,
        grid=(NP_PAD // blk,),
        in_specs=[
            pl.BlockSpec((blk, HC), lambda i: (i, 0)),
            pl.BlockSpec((blk, HC), lambda i: (i, 0)),
            pl.BlockSpec((1, HC), lambda i: (0, 0)),
        ],
        out_specs=pl.BlockSpec((blk, HC), lambda i: (i, 0)),
        out_shape=jax.ShapeDtypeStruct((NP_PAD, HC), jnp.float32),
    )(o0, o1, bias[None, :])


def _den_merge_body(d0_ref, d1_ref, o_ref):
    o_ref[...] = d0_ref[...] + d1_ref[...]


def _den_merge(d0, d1):
    return pl.pallas_call(
        _den_merge_body,
        out_shape=jax.ShapeDtypeStruct((NP_PAD, H), jnp.float32),
    )(d0, d1)


def _aggr1_body(h_ref, bt_ref, t_ref, smax_ref):
    s = h_ref[...] * t_ref[0, 0]
    bt = bt_ref[...]
    @pl.when(pl.program_id(0) == 0)
    def _():
        smax_ref[...] = jnp.full_like(smax_ref, -jnp.inf)
    for b in range(B):
        mb = jnp.max(jnp.where(bt == b, s, -jnp.inf), axis=0,
                     keepdims=True)
        smax_ref[b:b + 1, :] = jnp.maximum(smax_ref[b:b + 1, :], mb)


def _aggr2_body(h_ref, bt_ref, t_ref, smax_ref, den_ref):
    s = h_ref[...] * t_ref[0, 0]
    bt = bt_ref[...]
    onehot = (bt ==
              lax.broadcasted_iota(jnp.int32, (1, B), 1)).astype(jnp.float32)
    smax = smax_ref[...]
    smax = jnp.where(jnp.isfinite(smax), smax, 0.0)
    ex = jnp.exp(s - jnp.dot(onehot, smax, preferred_element_type=jnp.float32))
    @pl.when(pl.program_id(0) == 0)
    def _():
        den_ref[...] = jnp.zeros_like(den_ref)
    den_ref[...] += lax.dot_general(onehot, ex, (((0,), (0,)), ((), ())),
                                    preferred_element_type=jnp.float32)


def _aggr3_body(h_ref, bt_ref, t_ref, smax_ref, den_ref, w_ref, b_ref,
                out_ref, num_ref):
    x = h_ref[...]
    s = x * t_ref[0, 0]
    bt = bt_ref[...]
    onehot = (bt ==
              lax.broadcasted_iota(jnp.int32, (1, B), 1)).astype(jnp.float32)
    smax = smax_ref[...]
    smax = jnp.where(jnp.isfinite(smax), smax, 0.0)
    ex = jnp.exp(s - jnp.dot(onehot, smax, preferred_element_type=jnp.float32))
    den_n = jnp.dot(onehot, den_ref[...], preferred_element_type=jnp.float32)
    alpha = ex / (den_n + 1e-16)
    @pl.when(pl.program_id(0) == 0)
    def _():
        num_ref[...] = jnp.zeros_like(num_ref)
    num_ref[...] += lax.dot_general(onehot, alpha * x, (((0,), (0,)), ((), ())),
                                    preferred_element_type=jnp.float32)
    @pl.when(pl.program_id(0) == pl.num_programs(0) - 1)
    def _():
        out_ref[...] = jnp.dot(num_ref[...], w_ref[...],
                               preferred_element_type=jnp.float32) + b_ref[...]


def _aggr(h, batch, t, lin_w, lin_b):
    blk = 2048
    nc = lin_w.shape[1]
    t2 = t.reshape(1, 1)
    common = [
        pl.BlockSpec((blk, HC), lambda i: (i, 0)),
        pl.BlockSpec((blk, 1), lambda i: (i, 0)),
        pl.BlockSpec((1, 1), lambda i: (0, 0)),
    ]
    smax = pl.pallas_call(
        _aggr1_body,
        grid=(NP_PAD // blk,),
        in_specs=common,
        out_specs=pl.BlockSpec((B, HC), lambda i: (0, 0)),
        out_shape=jax.ShapeDtypeStruct((B, HC), jnp.float32),
    )(h, batch, t2)
    den = pl.pallas_call(
        _aggr2_body,
        grid=(NP_PAD // blk,),
        in_specs=common + [pl.BlockSpec((B, HC), lambda i: (0, 0))],
        out_specs=pl.BlockSpec((B, HC), lambda i: (0, 0)),
        out_shape=jax.ShapeDtypeStruct((B, HC), jnp.float32),
    )(h, batch, t2, smax)
    out = pl.pallas_call(
        _aggr3_body,
        grid=(NP_PAD // blk,),
        in_specs=common + [
            pl.BlockSpec((B, HC), lambda i: (0, 0)),
            pl.BlockSpec((B, HC), lambda i: (0, 0)),
            pl.BlockSpec((HC, nc), lambda i: (0, 0)),
            pl.BlockSpec((1, nc), lambda i: (0, 0)),
        ],
        out_specs=pl.BlockSpec((B, nc), lambda i: (0, 0)),
        out_shape=jax.ShapeDtypeStruct((B, nc), jnp.float32),
        scratch_shapes=[pltpu.VMEM((B, HC), jnp.float32)],
    )(h, batch, t2, smax, den, lin_w, lin_b[None, :])
    return out


def _sc_pass1_body(xl_hbm, xr_hbm, ee_hbm, src_hbm, dst_hbm, att_hbm, z_hbm,
                   ex_out, den_out0, den_out1,
                   idxs, idxd, xlg, xrg, eev, red1, red2, exb, attv, den_sh,
                   sem1, sem2, sem3):
    c = lax.axis_index("c")
    s = lax.axis_index("s")
    wid = c * NS_SC + s
    iot = lax.iota(jnp.int32, 16)

    pltpu.sync_copy(att_hbm, attv)
    pltpu.sync_copy(z_hbm.at[pl.ds(s * NPT, NPT), :],
                    den_sh.at[pl.ds(s * NPT, NPT), :])
    plsc.subcore_barrier()

    def chunk(i, carry):
        off = wid * EPT + i * CH
        pltpu.sync_copy(src_hbm.at[pl.ds(off, CH)], idxs.at[0])
        pltpu.sync_copy(dst_hbm.at[pl.ds(off, CH)], idxd.at[0])
        cp1 = pltpu.async_copy(xl_hbm.at[idxs.at[0]], xlg, sem1)
        cp2 = pltpu.async_copy(xr_hbm.at[idxd.at[0]], xrg, sem2)
        cp3 = pltpu.async_copy(ee_hbm.at[pl.ds(off, CH), :], eev, sem3)
        cp1.wait(); cp2.wait(); cp3.wait()
        for g in range(CH * 4):
            r, col = g // 4, (g % 4) * 16
            m = xlg[r, pl.ds(col, 16)] + xrg[r, pl.ds(col, 16)] \
                + eev[r, pl.ds(col, 16)]
            t = jnp.maximum(m, 0.2 * m) * attv[pl.ds(col, 16)]
            eev[r, pl.ds(col, 16)] = t
        for i2 in range(CH * 2):
            f0 = i2 * 32 + iot * 2
            v0 = plsc.load_gather(eev, [f0 >> 6, f0 & 63])
            v1 = plsc.load_gather(eev, [(f0 + 1) >> 6, (f0 + 1) & 63])
            red1[pl.ds(i2 * 16, 16)] = v0 + v1
        for i2 in range(CH):
            f0 = i2 * 32 + iot * 2
            red2[pl.ds(i2 * 16, 16)] = (plsc.load_gather(red1, [f0])
                                        + plsc.load_gather(red1, [f0 + 1]))
        for i2 in range(CH // 2):
            f0 = i2 * 32 + iot * 2
            v = plsc.load_gather(red2, [f0]) + plsc.load_gather(red2, [f0 + 1])
            ev = jnp.exp(v)
            flat = i2 * 16 + iot
            plsc.store_scatter(exb, [flat >> 3, flat & 7], ev)
        pltpu.sync_copy(exb, ex_out.at[pl.ds(off, CH), :])
        pltpu.sync_copy(exb, den_sh.at[idxd.at[0]], add=True)
        return carry

    lax.fori_loop(0, NCHUNK, chunk, 0)
    plsc.subcore_barrier()
    @pl.when(c == 0)
    def _():
        pltpu.sync_copy(den_sh.at[pl.ds(s * NPT, NPT), :],
                        den_out0.at[pl.ds(s * NPT, NPT), :])
    @pl.when(c == 1)
    def _():
        pltpu.sync_copy(den_sh.at[pl.ds(s * NPT, NPT), :],
                        den_out1.at[pl.ds(s * NPT, NPT), :])


def _sc_pass1(xl, xr, ee, src, dst, attf, zeros64):
    f = pl.kernel(
        _sc_pass1_body,
        out_type=(
            jax.ShapeDtypeStruct((E, H), jnp.float32),
            jax.ShapeDtypeStruct((NP_PAD, H), jnp.float32),
            jax.ShapeDtypeStruct((NP_PAD, H), jnp.float32),
        ),
        mesh=_SC_MESH,
        compiler_params=_SC_PARAMS,
        scratch_types=[
            pltpu.VMEM((1, CH), jnp.int32),
            pltpu.VMEM((1, CH), jnp.int32),
            pltpu.VMEM((CH, HC), jnp.float32),
            pltpu.VMEM((CH, HC), jnp.float32),
            pltpu.VMEM((CH, HC), jnp.float32),
            pltpu.VMEM((CH * 32,), jnp.float32),
            pltpu.VMEM((CH * 16,), jnp.float32),
            pltpu.VMEM((CH, H), jnp.float32),
            pltpu.VMEM((HC,), jnp.float32),
            pltpu.VMEM_SHARED((NP_PAD, H), jnp.float32),
            pltpu.SemaphoreType.DMA,
            pltpu.SemaphoreType.DMA,
            pltpu.SemaphoreType.DMA,
        ],
    )
    return f(xl, xr, ee, src, dst, attf, zeros64)


def _sc_pass2_body(xl_hbm, src_hbm, dst_hbm, ex_hbm, den_hbm, z_hbm,
                   out_p0, out_p1,
                   idxs, idxd, xlg, exv, alb, denv, out_sh,
                   sem1, sem2, sem3):
    c = lax.axis_index("c")
    s = lax.axis_index("s")
    wid = c * NS_SC + s
    iot = lax.iota(jnp.int32, 16)

    pltpu.sync_copy(den_hbm, denv)
    pltpu.sync_copy(z_hbm.at[pl.ds(s * NPT, NPT), :],
                    out_sh.at[pl.ds(s * NPT, NPT), :])
    plsc.subcore_barrier()

    def chunk(i, carry):
        off = wid * EPT + i * CH
        pltpu.sync_copy(src_hbm.at[pl.ds(off, CH)], idxs.at[0])
        pltpu.sync_copy(dst_hbm.at[pl.ds(off, CH)], idxd.at[0])
        cp1 = pltpu.async_copy(xl_hbm.at[idxs.at[0]], xlg, sem1)
        cp2 = pltpu.async_copy(ex_hbm.at[pl.ds(off, CH), :], exv, sem2)
        cp1.wait(); cp2.wait()
        for i2 in range(CH // 2):
            flat = i2 * 16 + iot
            erow = flat >> 3
            hh = flat & 7
            dv = plsc.load_gather(idxd, [iot * 0, erow])
            dval = plsc.load_gather(denv, [dv * H + hh])
            exval = plsc.load_gather(exv, [erow, hh])
            plsc.store_scatter(alb, [erow, hh], exval / (dval + 1e-16))
        for g in range(CH * 4):
            r, col = g // 4, (g % 4) * 16
            hvec = (col + iot) >> 3
            av = plsc.load_gather(alb, [iot * 0 + r, hvec])
            xlg[r, pl.ds(col, 16)] = xlg[r, pl.ds(col, 16)] * av
        pltpu.sync_copy(xlg, out_sh.at[idxd.at[0]], add=True)
        return carry

    lax.fori_loop(0, NCHUNK, chunk, 0)
    plsc.subcore_barrier()
    @pl.when(c == 0)
    def _():
        pltpu.sync_copy(out_sh.at[pl.ds(s * NPT, NPT), :],
                        out_p0.at[pl.ds(s * NPT, NPT), :])
    @pl.when(c == 1)
    def _():
        pltpu.sync_copy(out_sh.at[pl.ds(s * NPT, NPT), :],
                        out_p1.at[pl.ds(s * NPT, NPT), :])


def _sc_pass2(xl, src, dst, ex, den_flat, zeros64):
    f = pl.kernel(
        _sc_pass2_body,
        out_type=(
            jax.ShapeDtypeStruct((NP_PAD, HC), jnp.float32),
            jax.ShapeDtypeStruct((NP_PAD, HC), jnp.float32),
        ),
        mesh=_SC_MESH,
        compiler_params=_SC_PARAMS,
        scratch_types=[
            pltpu.VMEM((1, CH), jnp.int32),
            pltpu.VMEM((1, CH), jnp.int32),
            pltpu.VMEM((CH, HC), jnp.float32),
            pltpu.VMEM((CH, H), jnp.float32),
            pltpu.VMEM((CH, H), jnp.float32),
            pltpu.VMEM((NP_PAD * H,), jnp.float32),
            pltpu.VMEM_SHARED((NP_PAD, HC), jnp.float32),
            pltpu.SemaphoreType.DMA,
            pltpu.SemaphoreType.DMA,
            pltpu.SemaphoreType.DMA,
        ],
    )
    return f(xl, src, dst, ex, den_flat, zeros64)


def kernel(x, edge_index, edge_attr, batch, params):
    src, dst = edge_index[0], edge_index[1]
    zeros8 = jnp.zeros((NP_PAD, H), jnp.float32)
    zeros64 = jnp.zeros((NP_PAD, HC), jnp.float32)

    stx = _col_stats(x, 2000)
    ste = _col_stats(edge_attr, 8000)
    ean = _bn_ea(edge_attr, ste, params['en_g'], params['en_b'])

    p0 = params['layers'][0]
    xl, xr = _bn_mm1(x, stx, params['nn_g'], params['nn_b'],
                     p0['Wl'], p0['Wr'])

    o0 = o1 = None
    for li, p in enumerate(params['layers']):
        if li > 0:
            xl, xr = _merge_mm(o0, o1, params['layers'][li - 1]['bias'],
                               p['Wl'], p['Wr'])
        ee = _ee_mm(ean, p['We'])
        attf = p['att'].reshape(HC)
        ex, den0, den1 = _sc_pass1(xl, xr, ee, src, dst, attf, zeros8)
        den = _den_merge(den0, den1)
        den_flat = den.reshape(NP_PAD * H)
        o0, o1 = _sc_pass2(xl, src, dst, ex, den_flat, zeros64)

    h3 = _merge_h(o0, o1, params['layers'][-1]['bias'])
    batch_pad = jnp.concatenate(
        [batch, jnp.full((NP_PAD - N,), B, jnp.int32)]).reshape(NP_PAD, 1)
    return _aggr(h3, batch_pad, params['t'], params['lin_W'], params['lin_b'])
